# baseline (speedup 1.0000x reference)
import functools
import jax
import jax.numpy as jnp
from jax import lax
from jax.experimental import pallas as pl
from jax.experimental.pallas import tpu as pltpu, tpu_sc as plsc

NC, NS, L = 2, 16, 16
NW = NC * NS
NSLOT = 4096 * 12
SPW = NSLOT // NW
GR = 128
NSB = SPW // GR
OUTW = 768
NBUF = 4
LOOK = 2

_mesh = plsc.VectorSubcoreMesh(core_axis_name="c", subcore_axis_name="s")


@functools.partial(
    pl.kernel,
    out_type=jax.ShapeDtypeStruct((NSLOT, OUTW), jnp.float32),
    mesh=_mesh,
    scratch_types=[
        pltpu.VMEM((SPW * 7,), jnp.int32),
        pltpu.VMEM((7, SPW), jnp.int32),
        pltpu.VMEM((NBUF, GR, 128), jnp.float32),
        pltpu.VMEM((NBUF, GR, 64), jnp.float32),
        pltpu.SemaphoreType.DMA((NBUF,)),
        pltpu.SemaphoreType.DMA((NBUF,)),
        pltpu.SemaphoreType.DMA((NBUF,)),
        pltpu.SemaphoreType.DMA((NBUF,)),
        pltpu.SemaphoreType.DMA,
    ],
    compiler_params=pltpu.CompilerParams(
        use_tc_tiling_on_sc=False, needs_layout_passes=False),
)
def _embed(sp_hbm, mv_hbm, ab_hbm, it_hbm, ids_hbm, out_hbm,
           ids_v, idx_v, wbuf, nbuf, wgs, wws, ngs, nws, isem):
    wid = lax.axis_index("s") * NC + lax.axis_index("c")
    slot0 = wid * SPW
    pltpu.async_copy(ids_hbm.at[wid], ids_v, isem).wait()
    lane = lax.iota(jnp.int32, L)

    @pl.loop(0, SPW // L)
    def _build(g):
        base = g * L
        pos7 = (base + lane) * 7
        for c in range(7):
            idx_v[c, pl.ds(base, L)] = plsc.load_gather(ids_v, [pos7 + c])

    def _run_phase(tbl, rows, row0, col0, colstep, buf, gsems, wsems, width):
        def _gdesc(k, b):
            r = row0 + k // NSB
            sb = k % NSB
            return pltpu.make_async_copy(
                tbl.at[idx_v.at[r, pl.ds(sb * GR, GR)]],
                buf.at[b], gsems.at[b])

        def _wdesc(k, b):
            r = k // NSB
            sb = k % NSB
            return pltpu.make_async_copy(
                buf.at[b],
                out_hbm.at[pl.ds(slot0 + sb * GR, GR),
                           pl.ds(col0 + r * colstep, width)],
                wsems.at[b])

        for k in range(LOOK):
            _gdesc(k, k).start()

        @pl.loop(0, rows)
        def _step(k):
            b = lax.rem(k, NBUF)
            _gdesc(k, b).wait()
            _wdesc(k, b).start()
            g = k + LOOK

            @pl.when(g < rows)
            def _():
                bg = lax.rem(g, NBUF)

                @pl.when(g >= NBUF)
                def _():
                    _wdesc(g - NBUF, bg).wait()
                _gdesc(g, bg).start()

        for d in range(NBUF):
            k = rows - NBUF + d
            if k >= 0:
                _wdesc(k, k % NBUF).wait()

    _run_phase(sp_hbm, NSB, 0, 0, 128, wbuf, wgs, wws, 128)
    _run_phase(mv_hbm, 4 * NSB, 1, 128, 128, wbuf, wgs, wws, 128)
    _run_phase(ab_hbm, NSB, 5, 640, 64, nbuf, ngs, nws, 64)
    _run_phase(it_hbm, NSB, 6, 704, 64, nbuf, ngs, nws, 64)


def kernel(int_ids, species_table, move_table, ability_table, item_table):
    ids = int_ids.astype(jnp.int32).reshape(NW, SPW * 7)
    out = _embed(species_table, move_table, ability_table, item_table, ids)
    return out.reshape(4096, 12, 768)


# speedup vs baseline: 1.8615x; 1.8615x over previous
import functools
import jax
import jax.numpy as jnp
from jax import lax
from jax.experimental import pallas as pl
from jax.experimental.pallas import tpu as pltpu, tpu_sc as plsc

NC, NS, L = 2, 16, 16
NW = NC * NS
NB = 4096
NP = 12
BPW = NB // NW
NBUF = 4
IBUF = 2
LOOK = 2

_mesh = plsc.VectorSubcoreMesh(core_axis_name="c", subcore_axis_name="s")


@functools.partial(
    pl.kernel,
    out_type=jax.ShapeDtypeStruct((NP, NB, 768), jnp.float32),
    mesh=_mesh,
    scratch_types=[
        pltpu.VMEM((BPW * NP * 7,), jnp.int32),
        pltpu.VMEM((NP * 7, BPW), jnp.int32),
        pltpu.VMEM((NBUF, BPW, 128), jnp.float32),
        pltpu.VMEM((IBUF, BPW, 128), jnp.float32),
        pltpu.SemaphoreType.DMA((NBUF,)),
        pltpu.SemaphoreType.DMA((NBUF,)),
        pltpu.SemaphoreType.DMA((IBUF,)),
        pltpu.SemaphoreType.DMA,
    ],
    compiler_params=pltpu.CompilerParams(
        use_tc_tiling_on_sc=True, needs_layout_passes=False),
)
def _embed(sp_hbm, mv_hbm, ab_hbm, it_hbm, ids_hbm, out_hbm,
           ids_v, idx_v, ring, iring, gsems, wsems, isems, ssem):
    wid = lax.axis_index("s") * NC + lax.axis_index("c")
    bat0 = wid * BPW
    pltpu.async_copy(ids_hbm.at[wid], ids_v, ssem).wait()
    lane = lax.iota(jnp.int32, L)

    @pl.loop(0, BPW // L)
    def _build(g):
        base = g * L
        pos = (base + lane) * (NP * 7)
        for r in range(NP * 7):
            idx_v[r, pl.ds(base, L)] = plsc.load_gather(ids_v, [pos + r])

    def _gdesc(tbl, r, b):
        return pltpu.make_async_copy(
            tbl.at[idx_v.at[r]], ring.at[b], gsems.at[b])

    def _wdesc(p, col, b):
        return pltpu.make_async_copy(
            ring.at[b],
            out_hbm.at[p, pl.ds(bat0, BPW), pl.ds(col * 128, 128)],
            wsems.at[b])

    def _run_phase(steps, gfn, wfn):
        for k in range(LOOK):
            gfn(k, k).start()

        @pl.loop(0, steps)
        def _step(k):
            b = lax.rem(k, NBUF)
            gfn(k, b).wait()
            p, col = wfn(k)
            _wdesc(p, col, b).start()
            g = k + LOOK

            @pl.when(g < steps)
            def _():
                bg = lax.rem(g, NBUF)

                @pl.when(g >= NBUF)
                def _():
                    pg, colg = wfn(g - NBUF)
                    _wdesc(pg, colg, bg).wait()
                gfn(g, bg).start()

        for d in range(NBUF):
            k = steps - NBUF + d
            p, col = wfn(k)
            _wdesc(p, col, k % NBUF).wait()

    _run_phase(NP,
               lambda k, b: _gdesc(sp_hbm, k * 7, b),
               lambda k: (k, 0))

    _run_phase(4 * NP,
               lambda k, b: _gdesc(mv_hbm, (k // 4) * 7 + 1 + lax.rem(k, 4), b),
               lambda k: (k // 4, 1 + lax.rem(k, 4)))

    def _abit_g(k, b):
        return _gdesc(ab_hbm, k * 7 + 5, b)

    for k in range(LOOK):
        _abit_g(k, k).start()
    for k in range(IBUF):
        pltpu.make_async_copy(
            it_hbm.at[idx_v.at[k * 7 + 6]], iring.at[k], isems.at[k]).start()

    @pl.loop(0, NP)
    def _abit(k):
        b = lax.rem(k, NBUF)
        ib = lax.rem(k, IBUF)
        _abit_g(k, b).wait()
        pltpu.make_async_copy(
            it_hbm.at[idx_v.at[k * 7 + 6]], iring.at[ib], isems.at[ib]).wait()
        @pl.loop(0, BPW)
        def _merge(r):
            for q in range(4):
                ring[b, r, pl.ds(64 + q * L, L)] = iring[ib, r, pl.ds(64 + q * L, L)]
        _wdesc(k, 5, b).start()
        g = k + LOOK

        @pl.when(g < NP)
        def _():
            bg = lax.rem(g, NBUF)
            pg, colg = (g - NBUF, 5)

            @pl.when(g >= NBUF)
            def _():
                _wdesc(pg, colg, bg).wait()
            _abit_g(g, bg).start()
            ig = lax.rem(g, IBUF)
            pltpu.make_async_copy(
                it_hbm.at[idx_v.at[g * 7 + 6]], iring.at[ig],
                isems.at[ig]).start()

    for d in range(NBUF):
        k = NP - NBUF + d
        _wdesc(k, 5, k % NBUF).wait()


def kernel(int_ids, species_table, move_table, ability_table, item_table):
    ids = int_ids.astype(jnp.int32).reshape(NW, BPW * NP * 7)
    ab_p = jnp.pad(ability_table, ((0, 0), (0, 64)))
    it_p = jnp.pad(item_table, ((0, 0), (64, 0)))
    out = _embed(species_table, move_table, ab_p, it_p, ids)
    return jnp.transpose(out, (1, 0, 2))
